# baseline (speedup 1.0000x reference)
import functools

import jax
import jax.numpy as jnp
from jax import lax
from jax.experimental import pallas as pl
from jax.experimental.pallas import tpu as pltpu
from jax.experimental.pallas import tpu_sc as plsc

N = 10000
D = 128
E = 320000
ALPHA = 0.1
K_PROP = 10

NC = 2
DH = D // NC
NT = 16
NP = 10240
NPT = NP // NT
EPT = E // NT
CH = 64
NBUF = 4
IBUF = 8
NCH = 320
EPT_P = NCH * CH
DUMMY_SRC = N
DUMMY_DST = N + 1
UCH = 32
NUC = NPT // UCH
L = 16
HR = NP // DH


def _mm_body(x_ref, w_ref, b_ref, o_ref):
    o_ref[...] = (
        jnp.dot(x_ref[...], w_ref[...], preferred_element_type=jnp.float32)
        + b_ref[...]
    )


def _h0_matmul(x, wt, b2):
    return pl.pallas_call(
        _mm_body,
        grid=(10,),
        in_specs=[
            pl.BlockSpec((N // 10, D), lambda i: (i, 0)),
            pl.BlockSpec((D, D), lambda i: (0, 0)),
            pl.BlockSpec((1, D), lambda i: (0, 0)),
        ],
        out_specs=pl.BlockSpec((N // 10, D), lambda i: (i, 0)),
        out_shape=jax.ShapeDtypeStruct((N, D), jnp.float32),
    )(x, wt, b2)


def _bcast_row(scal_ref, row):
    idx = jnp.full((L,), row, dtype=jnp.int32)
    return plsc.load_gather(scal_ref, [idx])


def _rsqrt16(d):
    i = plsc.bitcast(d, jnp.int32)
    y = plsc.bitcast(jnp.int32(0x5F3759DF) - (i >> 1), jnp.float32)
    for _ in range(3):
        y = y * (1.5 - 0.5 * d * y * y)
    return y


HBM_SLOTS = (0, 2, 4, 6)


def _sc_body(h0_hbm, idxp_hbm, out_hbm, c_hbm, p_hbm,
             S_sh, p_sh,
             pairbuf, rowbuf, hist_v, tmp_v, u_v, y_v, rd_v,
             sbuf, pbuf, cbuf, zbuf,
             gs, ss, ip, us, up, uc, uw, uw2, uz):
    cid = lax.axis_index("c")
    wid = lax.axis_index("s")
    nbase = wid * NPT
    coff = cid * NP

    @pl.loop(0, HR)
    def _(g):
        for i in range(DH // L):
            hist_v[g, pl.ds(i * L, L)] = jnp.zeros((L,), jnp.float32)

    @pl.loop(0, UCH)
    def _(r):
        for i in range(DH // L):
            zbuf[r, pl.ds(i * L, L)] = jnp.zeros((L,), jnp.float32)

    ones = jnp.ones((L,), jnp.float32)

    @pl.loop(0, NCH // IBUF)
    def _(g):
        pltpu.sync_copy(idxp_hbm.at[wid, pl.ds(g * IBUF, IBUF)], pairbuf)
        for q in range(IBUF):
            for i in range(CH // L):
                sv = pairbuf[q, 0, pl.ds(i * L, L)]
                plsc.addupdate_scatter(hist_v, [sv >> 6, sv & 63], ones)

    pltpu.sync_copy(hist_v, S_sh.at[pl.ds(wid * HR, HR)])
    plsc.subcore_barrier()

    @pl.loop(0, NPT // L)
    def _(g):
        u_v[pl.ds(g * L, L)] = jnp.zeros((L,), jnp.float32)

    nrow = NPT // DH
    for tt in range(NT):
        pltpu.sync_copy(S_sh.at[pl.ds(tt * HR + wid * nrow, nrow)], tmp_v)

        @pl.loop(0, NPT // L)
        def _(g):
            s = pl.ds(g * L, L)
            u_v[s] = u_v[s] + tmp_v[g // (DH // L), pl.ds((g % (DH // L)) * L, L)]

    plsc.subcore_barrier()

    @pl.loop(0, NPT // L)
    def _(g):
        s = pl.ds(g * L, L)
        deg = u_v[s] + 1.0
        y = _rsqrt16(deg)
        y_v[s] = y
        rd_v[s] = y * deg
        u_v[s] = (1.0 - ALPHA) * y * y

    for ch in range(NUC):
        rb = nbase + ch * UCH
        pltpu.sync_copy(h0_hbm.at[pl.ds(coff + rb, UCH)], pbuf.at[0])

        @pl.loop(0, UCH)
        def _(r):
            yb = _bcast_row(y_v, ch * UCH + r)
            for i in range(DH // L):
                s = pl.ds(i * L, L)
                h0r = pbuf[0, r, s]
                cbuf[0, r, s] = (ALPHA * yb) * h0r
                pbuf[0, r, s] = yb * h0r

        pltpu.sync_copy(cbuf.at[0], c_hbm.at[pl.ds(coff + rb, UCH)])
        pltpu.sync_copy(pbuf.at[0], p_sh.at[pl.ds(rb, UCH)])
        pltpu.sync_copy(pbuf.at[0], p_hbm.at[pl.ds(coff + rb, UCH)])
        pltpu.sync_copy(zbuf, S_sh.at[pl.ds(rb, UCH)])

    plsc.subcore_barrier()

    def i_start(j, b8):
        pltpu.async_copy(idxp_hbm.at[wid, j], pairbuf.at[b8], ip[b8])

    def i_wait(j, b8):
        pltpu.make_async_copy(idxp_hbm.at[wid, j], pairbuf.at[b8],
                              ip[b8]).wait()

    coffv = jnp.full((L,), coff, dtype=jnp.int32)

    def add_coff(b8):
        for i in range(CH // L):
            s = pl.ds(i * L, L)
            pairbuf[b8, 1, s] = pairbuf[b8, 1, s] + coffv

    def g_src(b8):
        if b8 % IBUF in HBM_SLOTS:
            return p_hbm.at[pairbuf.at[b8, 1]]
        return p_sh.at[pairbuf.at[b8, 1]]

    def g_start(b, b8):
        return pltpu.async_copy(g_src(b8), rowbuf.at[b], gs[b])

    def g_wait(b, b8):
        pltpu.make_async_copy(g_src(b8), rowbuf.at[b], gs[b]).wait()

    def s_start(b, b8):
        return pltpu.async_copy(rowbuf.at[b], S_sh.at[pairbuf.at[b8, 0]],
                                ss[b], add=True)

    def s_wait(b, b8):
        pltpu.make_async_copy(rowbuf.at[b], S_sh.at[pairbuf.at[b8, 0]],
                              ss[b]).wait()

    def upd_prefetch(ch, b):
        rb = nbase + ch * UCH
        pltpu.async_copy(S_sh.at[pl.ds(rb, UCH)], sbuf.at[b], us[b])
        pltpu.async_copy(p_sh.at[pl.ds(rb, UCH)], pbuf.at[b], up[b])
        pltpu.async_copy(c_hbm.at[pl.ds(coff + rb, UCH)], cbuf.at[b], uc[b])

    def upd_wait_in(ch, b):
        rb = nbase + ch * UCH
        pltpu.make_async_copy(S_sh.at[pl.ds(rb, UCH)], sbuf.at[b],
                              us[b]).wait()
        pltpu.make_async_copy(p_sh.at[pl.ds(rb, UCH)], pbuf.at[b],
                              up[b]).wait()
        pltpu.make_async_copy(c_hbm.at[pl.ds(coff + rb, UCH)], cbuf.at[b],
                              uc[b]).wait()

    def upd_wait_w(ch, b):
        rb = nbase + ch * UCH
        pltpu.make_async_copy(pbuf.at[b], p_sh.at[pl.ds(rb, UCH)],
                              uw[b]).wait()
        pltpu.make_async_copy(pbuf.at[b], p_hbm.at[pl.ds(coff + rb, UCH)],
                              uw2[b]).wait()

    @pl.loop(0, K_PROP)
    def _(k):
        for jp in range(IBUF - 2):
            i_start(jp, jp)
        i_wait(0, 0)
        if 0 in HBM_SLOTS:
            add_coff(0)
        g_start(0, 0)
        i_wait(1, 1)
        if 1 in HBM_SLOTS:
            add_coff(1)
        g_start(1, 1)

        @pl.loop(0, NCH // IBUF)
        def _(jj):
            for t in range(IBUF):
                j = jj * IBUF + t
                b = t % NBUF
                g_wait(b, t)
                s_start(b, t)

                @pl.when(j >= 2)
                def _():
                    s_wait((t - 2) % NBUF, (t - 2) % IBUF)

                @pl.when(j + 2 < NCH)
                def _():
                    i_wait(j + 2, (t + 2) % IBUF)
                    if (t + 2) % IBUF in HBM_SLOTS:
                        add_coff((t + 2) % IBUF)
                    g_start((t + 2) % NBUF, (t + 2) % IBUF)

                @pl.when(j + 6 < NCH)
                def _():
                    i_start(j + 6, (t + 6) % IBUF)

        s_wait((NCH - 2) % NBUF, (NCH - 2) % IBUF)
        s_wait((NCH - 1) % NBUF, (NCH - 1) % IBUF)

        plsc.subcore_barrier()

        upd_prefetch(0, 0)
        for ch in range(NUC):
            b = ch % 2
            upd_wait_in(ch, b)
            if ch + 1 < NUC:
                if ch >= 1:
                    upd_wait_w(ch - 1, 1 - b)
                upd_prefetch(ch + 1, 1 - b)

            @pl.loop(0, UCH)
            def _(r):
                ub = _bcast_row(u_v, ch * UCH + r)
                for i in range(DH // L):
                    s = pl.ds(i * L, L)
                    pbuf[b, r, s] = (ub * (sbuf[b, r, s] + pbuf[b, r, s])
                                     + cbuf[b, r, s])

            rb = nbase + ch * UCH
            pltpu.async_copy(pbuf.at[b], p_sh.at[pl.ds(rb, UCH)], uw[b])
            pltpu.async_copy(pbuf.at[b], p_hbm.at[pl.ds(coff + rb, UCH)],
                             uw2[b])
            pltpu.async_copy(zbuf, S_sh.at[pl.ds(rb, UCH)], uz)

        upd_wait_w(NUC - 2, 1 - (NUC - 1) % 2)
        upd_wait_w(NUC - 1, (NUC - 1) % 2)
        for ch in range(NUC):
            rb = nbase + ch * UCH
            pltpu.make_async_copy(zbuf, S_sh.at[pl.ds(rb, UCH)], uz).wait()

        plsc.subcore_barrier()

    for ch in range(NUC):
        rb = nbase + ch * UCH
        pltpu.sync_copy(p_sh.at[pl.ds(rb, UCH)], pbuf.at[0])

        @pl.loop(0, UCH)
        def _(r):
            rdb = _bcast_row(rd_v, ch * UCH + r)
            for i in range(DH // L):
                s = pl.ds(i * L, L)
                pbuf[0, r, s] = rdb * pbuf[0, r, s]

        pltpu.sync_copy(pbuf.at[0], out_hbm.at[pl.ds(coff + rb, UCH)])


@functools.partial(
    pl.kernel,
    out_type=(
        jax.ShapeDtypeStruct((NC * NP, DH), jnp.float32),
        jax.ShapeDtypeStruct((NC * NP, DH), jnp.float32),
        jax.ShapeDtypeStruct((NC * NP, DH), jnp.float32),
    ),
    mesh=plsc.VectorSubcoreMesh(
        core_axis_name="c", subcore_axis_name="s", num_cores=NC),
    compiler_params=pltpu.CompilerParams(
        needs_layout_passes=False, use_tc_tiling_on_sc=False),
    scratch_types=[
        pltpu.VMEM_SHARED((NP, DH), jnp.float32),
        pltpu.VMEM_SHARED((NP, DH), jnp.float32),
        pltpu.VMEM((IBUF, 2, CH), jnp.int32),
        pltpu.VMEM((NBUF, CH, DH), jnp.float32),
        pltpu.VMEM((HR, DH), jnp.float32),
        pltpu.VMEM((NPT // DH, DH), jnp.float32),
        pltpu.VMEM((NPT,), jnp.float32),
        pltpu.VMEM((NPT,), jnp.float32),
        pltpu.VMEM((NPT,), jnp.float32),
        pltpu.VMEM((2, UCH, DH), jnp.float32),
        pltpu.VMEM((2, UCH, DH), jnp.float32),
        pltpu.VMEM((2, UCH, DH), jnp.float32),
        pltpu.VMEM((UCH, DH), jnp.float32),
        [pltpu.SemaphoreType.DMA] * NBUF,
        [pltpu.SemaphoreType.DMA] * NBUF,
        [pltpu.SemaphoreType.DMA] * IBUF,
        [pltpu.SemaphoreType.DMA] * 2,
        [pltpu.SemaphoreType.DMA] * 2,
        [pltpu.SemaphoreType.DMA] * 2,
        [pltpu.SemaphoreType.DMA] * 2,
        [pltpu.SemaphoreType.DMA] * 2,
        pltpu.SemaphoreType.DMA,
    ],
)
def _sc_propagate(h0_hbm, idxp_hbm, out_hbm, c_hbm, p_hbm, *rest):
    _sc_body(h0_hbm, idxp_hbm, out_hbm, c_hbm, p_hbm, *rest)


def kernel(x, edge_index, W, b):
    src = edge_index[0].astype(jnp.int32)
    dst = edge_index[1].astype(jnp.int32)
    h0 = _h0_matmul(x, W.T, b.reshape(1, D))
    h0p = jnp.pad(h0, ((0, NP - N), (0, 0)))
    h0f = jnp.concatenate([h0p[:, :DH], h0p[:, DH:]], axis=0)
    srcp = jnp.pad(src.reshape(NT, EPT), ((0, 0), (0, EPT_P - EPT)),
                   constant_values=DUMMY_SRC).reshape(NT, NCH, 1, CH)
    dstp = jnp.pad(dst.reshape(NT, EPT), ((0, 0), (0, EPT_P - EPT)),
                   constant_values=DUMMY_DST).reshape(NT, NCH, 1, CH)
    idxp = jnp.concatenate([srcp, dstp], axis=2)
    out, _, _ = _sc_propagate(h0f, idxp)
    return jnp.concatenate([out[:N], out[NP:NP + N]], axis=1)


# speedup vs baseline: 1.4175x; 1.4175x over previous
import functools

import jax
import jax.numpy as jnp
from jax import lax
from jax.experimental import pallas as pl
from jax.experimental.pallas import tpu as pltpu
from jax.experimental.pallas import tpu_sc as plsc

N = 10000
D = 128
E = 320000
ALPHA = 0.1
K_PROP = 10

NC = 2
DH = D // NC
NT = 16
NP = 10240
NPT = NP // NT
EPT = E // NT
CH = 128
NBUF = 2
IBUF = 8
NCH = 160
EPT_P = NCH * CH
DUMMY_SRC = N
DUMMY_DST = N + 1
UCH = 32
NUC = NPT // UCH
L = 16
HR = NP // DH


def _mm_body(x_ref, w_ref, b_ref, o_ref):
    o_ref[...] = (
        jnp.dot(x_ref[...], w_ref[...], preferred_element_type=jnp.float32)
        + b_ref[...]
    )


def _h0_matmul(x, wt, b2):
    return pl.pallas_call(
        _mm_body,
        grid=(10,),
        in_specs=[
            pl.BlockSpec((N // 10, D), lambda i: (i, 0)),
            pl.BlockSpec((D, D), lambda i: (0, 0)),
            pl.BlockSpec((1, D), lambda i: (0, 0)),
        ],
        out_specs=pl.BlockSpec((N // 10, D), lambda i: (i, 0)),
        out_shape=jax.ShapeDtypeStruct((N, D), jnp.float32),
    )(x, wt, b2)


def _bcast_row(scal_ref, row):
    idx = jnp.full((L,), row, dtype=jnp.int32)
    return plsc.load_gather(scal_ref, [idx])


def _rsqrt16(d):
    i = plsc.bitcast(d, jnp.int32)
    y = plsc.bitcast(jnp.int32(0x5F3759DF) - (i >> 1), jnp.float32)
    for _ in range(3):
        y = y * (1.5 - 0.5 * d * y * y)
    return y


def _sc_body(h0_hbm, idxp_hbm, out_hbm, c_hbm,
             S_sh, p_sh,
             pairbuf, rowbuf, hist_v, tmp_v, u_v, y_v, rd_v,
             sbuf, pbuf, cbuf, zbuf,
             gs, ss, ip, us, up, uc, uw, uz):
    cid = lax.axis_index("c")
    wid = lax.axis_index("s")
    nbase = wid * NPT
    coff = cid * NP

    @pl.loop(0, HR)
    def _(g):
        for i in range(DH // L):
            hist_v[g, pl.ds(i * L, L)] = jnp.zeros((L,), jnp.float32)

    @pl.loop(0, UCH)
    def _(r):
        for i in range(DH // L):
            zbuf[r, pl.ds(i * L, L)] = jnp.zeros((L,), jnp.float32)

    ones = jnp.ones((L,), jnp.float32)

    @pl.loop(0, NCH // IBUF)
    def _(g):
        pltpu.sync_copy(idxp_hbm.at[wid, pl.ds(g * IBUF, IBUF)], pairbuf)
        for q in range(IBUF):
            for i in range(CH // L):
                sv = pairbuf[q, 0, pl.ds(i * L, L)]
                plsc.addupdate_scatter(hist_v, [sv >> 6, sv & 63], ones)

    pltpu.sync_copy(hist_v, S_sh.at[pl.ds(wid * HR, HR)])
    plsc.subcore_barrier()

    @pl.loop(0, NPT // L)
    def _(g):
        u_v[pl.ds(g * L, L)] = jnp.zeros((L,), jnp.float32)

    nrow = NPT // DH
    for tt in range(NT):
        pltpu.sync_copy(S_sh.at[pl.ds(tt * HR + wid * nrow, nrow)], tmp_v)

        @pl.loop(0, NPT // L)
        def _(g):
            s = pl.ds(g * L, L)
            u_v[s] = u_v[s] + tmp_v[g // (DH // L), pl.ds((g % (DH // L)) * L, L)]

    plsc.subcore_barrier()

    @pl.loop(0, NPT // L)
    def _(g):
        s = pl.ds(g * L, L)
        deg = u_v[s] + 1.0
        y = _rsqrt16(deg)
        y_v[s] = y
        rd_v[s] = y * deg
        u_v[s] = (1.0 - ALPHA) * y * y

    for ch in range(NUC):
        rb = nbase + ch * UCH
        pltpu.sync_copy(h0_hbm.at[pl.ds(coff + rb, UCH)], pbuf.at[0])

        @pl.loop(0, UCH)
        def _(r):
            yb = _bcast_row(y_v, ch * UCH + r)
            for i in range(DH // L):
                s = pl.ds(i * L, L)
                h0r = pbuf[0, r, s]
                cbuf[0, r, s] = (ALPHA * yb) * h0r
                pbuf[0, r, s] = yb * h0r

        pltpu.sync_copy(cbuf.at[0], c_hbm.at[pl.ds(coff + rb, UCH)])
        pltpu.sync_copy(pbuf.at[0], p_sh.at[pl.ds(rb, UCH)])
        pltpu.sync_copy(zbuf, S_sh.at[pl.ds(rb, UCH)])

    plsc.subcore_barrier()

    def i_start(j, b8):
        pltpu.async_copy(idxp_hbm.at[wid, j], pairbuf.at[b8], ip[b8])

    def i_wait(j, b8):
        pltpu.make_async_copy(idxp_hbm.at[wid, j], pairbuf.at[b8],
                              ip[b8]).wait()

    def g_start(b, b8):
        return pltpu.async_copy(p_sh.at[pairbuf.at[b8, 1]], rowbuf.at[b],
                                gs[b])

    def g_wait(b, b8):
        pltpu.make_async_copy(p_sh.at[pairbuf.at[b8, 1]], rowbuf.at[b],
                              gs[b]).wait()

    def s_start(b, b8):
        return pltpu.async_copy(rowbuf.at[b], S_sh.at[pairbuf.at[b8, 0]],
                                ss[b], add=True)

    def s_wait(b, b8):
        pltpu.make_async_copy(rowbuf.at[b], S_sh.at[pairbuf.at[b8, 0]],
                              ss[b]).wait()

    def upd_prefetch(ch, b):
        rb = nbase + ch * UCH
        pltpu.async_copy(S_sh.at[pl.ds(rb, UCH)], sbuf.at[b], us[b])
        pltpu.async_copy(p_sh.at[pl.ds(rb, UCH)], pbuf.at[b], up[b])
        pltpu.async_copy(c_hbm.at[pl.ds(coff + rb, UCH)], cbuf.at[b], uc[b])

    def upd_wait_in(ch, b):
        rb = nbase + ch * UCH
        pltpu.make_async_copy(S_sh.at[pl.ds(rb, UCH)], sbuf.at[b],
                              us[b]).wait()
        pltpu.make_async_copy(p_sh.at[pl.ds(rb, UCH)], pbuf.at[b],
                              up[b]).wait()
        pltpu.make_async_copy(c_hbm.at[pl.ds(coff + rb, UCH)], cbuf.at[b],
                              uc[b]).wait()

    def upd_wait_w(ch, b):
        rb = nbase + ch * UCH
        pltpu.make_async_copy(pbuf.at[b], p_sh.at[pl.ds(rb, UCH)],
                              uw[b]).wait()

    @pl.loop(0, K_PROP)
    def _(k):
        def slot(j, b, b8):
            g_wait(b, b8)
            pltpu.make_async_copy(rowbuf.at[1 - b],
                                  S_sh.at[pairbuf.at[(b8 - 1) % IBUF, 0]],
                                  ss[1 - b]).wait()
            if isinstance(j, int):
                nxt = j + 1 < NCH
            else:
                nxt = True
            if nxt:
                i_wait(j + 1, (b8 + 1) % IBUF)
                g_start(1 - b, (b8 + 1) % IBUF)
            s_start(b, b8)
            if isinstance(j, int):
                if j + 7 < NCH:
                    i_start(j + 7, (b8 + 7) % IBUF)
            else:
                i_start(j + 7, (b8 + 7) % IBUF)

        for jp in range(IBUF - 1):
            i_start(jp, jp)
        i_wait(0, 0)
        g_start(0, 0)
        g_wait(0, 0)
        i_wait(1, 1)
        g_start(1, 1)
        s_start(0, 0)
        i_start(7, 7)

        @pl.loop(0, (NCH - 8) // IBUF)
        def _(jj):
            for t in range(IBUF):
                j = jj * IBUF + t + 1
                slot(j, (1 + t) % NBUF, (1 + t) % IBUF)

        for j in range(NCH - 7, NCH):
            slot(j, j % NBUF, j % IBUF)

        s_wait((NCH - 1) % NBUF, (NCH - 1) % IBUF)

        plsc.subcore_barrier()

        upd_prefetch(0, 0)
        for ch in range(NUC):
            b = ch % 2
            upd_wait_in(ch, b)
            if ch + 1 < NUC:
                if ch >= 1:
                    upd_wait_w(ch - 1, 1 - b)
                upd_prefetch(ch + 1, 1 - b)

            @pl.loop(0, UCH)
            def _(r):
                ub = _bcast_row(u_v, ch * UCH + r)
                for i in range(DH // L):
                    s = pl.ds(i * L, L)
                    pbuf[b, r, s] = (ub * (sbuf[b, r, s] + pbuf[b, r, s])
                                     + cbuf[b, r, s])

            rb = nbase + ch * UCH
            pltpu.async_copy(pbuf.at[b], p_sh.at[pl.ds(rb, UCH)], uw[b])
            pltpu.async_copy(zbuf, S_sh.at[pl.ds(rb, UCH)], uz)

        upd_wait_w(NUC - 2, 1 - (NUC - 1) % 2)
        upd_wait_w(NUC - 1, (NUC - 1) % 2)
        for ch in range(NUC):
            rb = nbase + ch * UCH
            pltpu.make_async_copy(zbuf, S_sh.at[pl.ds(rb, UCH)], uz).wait()

        plsc.subcore_barrier()

    for ch in range(NUC):
        rb = nbase + ch * UCH
        pltpu.sync_copy(p_sh.at[pl.ds(rb, UCH)], pbuf.at[0])

        @pl.loop(0, UCH)
        def _(r):
            rdb = _bcast_row(rd_v, ch * UCH + r)
            for i in range(DH // L):
                s = pl.ds(i * L, L)
                pbuf[0, r, s] = rdb * pbuf[0, r, s]

        pltpu.sync_copy(pbuf.at[0], out_hbm.at[pl.ds(coff + rb, UCH)])


@functools.partial(
    pl.kernel,
    out_type=(
        jax.ShapeDtypeStruct((NC * NP, DH), jnp.float32),
        jax.ShapeDtypeStruct((NC * NP, DH), jnp.float32),
    ),
    mesh=plsc.VectorSubcoreMesh(
        core_axis_name="c", subcore_axis_name="s", num_cores=NC),
    compiler_params=pltpu.CompilerParams(
        needs_layout_passes=False, use_tc_tiling_on_sc=False),
    scratch_types=[
        pltpu.VMEM_SHARED((NP, DH), jnp.float32),
        pltpu.VMEM_SHARED((NP, DH), jnp.float32),
        pltpu.VMEM((IBUF, 2, CH), jnp.int32),
        pltpu.VMEM((NBUF, CH, DH), jnp.float32),
        pltpu.VMEM((HR, DH), jnp.float32),
        pltpu.VMEM((NPT // DH, DH), jnp.float32),
        pltpu.VMEM((NPT,), jnp.float32),
        pltpu.VMEM((NPT,), jnp.float32),
        pltpu.VMEM((NPT,), jnp.float32),
        pltpu.VMEM((2, UCH, DH), jnp.float32),
        pltpu.VMEM((2, UCH, DH), jnp.float32),
        pltpu.VMEM((2, UCH, DH), jnp.float32),
        pltpu.VMEM((UCH, DH), jnp.float32),
        [pltpu.SemaphoreType.DMA] * NBUF,
        [pltpu.SemaphoreType.DMA] * NBUF,
        [pltpu.SemaphoreType.DMA] * IBUF,
        [pltpu.SemaphoreType.DMA] * 2,
        [pltpu.SemaphoreType.DMA] * 2,
        [pltpu.SemaphoreType.DMA] * 2,
        [pltpu.SemaphoreType.DMA] * 2,
        pltpu.SemaphoreType.DMA,
    ],
)
def _sc_propagate(h0_hbm, idxp_hbm, out_hbm, c_hbm, *rest):
    _sc_body(h0_hbm, idxp_hbm, out_hbm, c_hbm, *rest)


def kernel(x, edge_index, W, b):
    src = edge_index[0].astype(jnp.int32)
    dst = edge_index[1].astype(jnp.int32)
    h0 = _h0_matmul(x, W.T, b.reshape(1, D))
    h0p = jnp.pad(h0, ((0, NP - N), (0, 0)))
    h0f = jnp.concatenate([h0p[:, :DH], h0p[:, DH:]], axis=0)
    srcp = jnp.pad(src.reshape(NT, EPT), ((0, 0), (0, EPT_P - EPT)),
                   constant_values=DUMMY_SRC).reshape(NT, NCH, 1, CH)
    dstp = jnp.pad(dst.reshape(NT, EPT), ((0, 0), (0, EPT_P - EPT)),
                   constant_values=DUMMY_DST).reshape(NT, NCH, 1, CH)
    idxp = jnp.concatenate([srcp, dstp], axis=2)
    out, _ = _sc_propagate(h0f, idxp)
    return jnp.concatenate([out[:N], out[NP:NP + N]], axis=1)


# speedup vs baseline: 1.6166x; 1.1404x over previous
import functools

import jax
import jax.numpy as jnp
from jax import lax
from jax.experimental import pallas as pl
from jax.experimental.pallas import tpu as pltpu
from jax.experimental.pallas import tpu_sc as plsc

N = 10000
D = 128
E = 320000
ALPHA = 0.1
K_PROP = 10

NC = 2
DH = D // NC
NT = 16
NP = 10240
NPT = NP // NT
EPT = E // NT
CH = 64
NBUF = 4
IBUF = 8
NCH = 320
EPT_P = NCH * CH
DUMMY_SRC = N
DUMMY_DST = N + 1
UCH = 32
NUC = NPT // UCH
L = 16
HR = NP // DH


def _mm_body(x_ref, w_ref, b_ref, o_ref):
    o_ref[...] = (
        jnp.dot(x_ref[...], w_ref[...], preferred_element_type=jnp.float32)
        + b_ref[...]
    )


def _h0_matmul(x, wt, b2):
    return pl.pallas_call(
        _mm_body,
        grid=(10,),
        in_specs=[
            pl.BlockSpec((N // 10, D), lambda i: (i, 0)),
            pl.BlockSpec((D, D), lambda i: (0, 0)),
            pl.BlockSpec((1, D), lambda i: (0, 0)),
        ],
        out_specs=pl.BlockSpec((N // 10, D), lambda i: (i, 0)),
        out_shape=jax.ShapeDtypeStruct((N, D), jnp.float32),
    )(x, wt, b2)


def _bcast_row(scal_ref, row):
    idx = jnp.full((L,), row, dtype=jnp.int32)
    return plsc.load_gather(scal_ref, [idx])


def _rsqrt16(d):
    i = plsc.bitcast(d, jnp.int32)
    y = plsc.bitcast(jnp.int32(0x5F3759DF) - (i >> 1), jnp.float32)
    for _ in range(3):
        y = y * (1.5 - 0.5 * d * y * y)
    return y


def _sc_body(h0_hbm, idxp_hbm, out_hbm, c_hbm,
             S_sh, p_sh,
             pairbuf, rowbuf, hist_v, tmp_v, u_v, y_v, rd_v,
             sbuf, pbuf, cbuf, zbuf,
             gs, ss, ip, us, up, uc, uw, uz):
    cid = lax.axis_index("c")
    wid = lax.axis_index("s")
    nbase = wid * NPT
    coff = cid * NP

    @pl.loop(0, HR)
    def _(g):
        for i in range(DH // L):
            hist_v[g, pl.ds(i * L, L)] = jnp.zeros((L,), jnp.float32)

    @pl.loop(0, UCH)
    def _(r):
        for i in range(DH // L):
            zbuf[r, pl.ds(i * L, L)] = jnp.zeros((L,), jnp.float32)

    ones = jnp.ones((L,), jnp.float32)

    @pl.loop(0, NCH // IBUF)
    def _(g):
        pltpu.sync_copy(idxp_hbm.at[wid, pl.ds(g * IBUF, IBUF)], pairbuf)
        for q in range(IBUF):
            for i in range(CH // L):
                sv = pairbuf[q, 0, pl.ds(i * L, L)]
                plsc.addupdate_scatter(hist_v, [sv >> 6, sv & 63], ones)

    pltpu.sync_copy(hist_v, S_sh.at[pl.ds(wid * HR, HR)])
    plsc.subcore_barrier()

    @pl.loop(0, NPT // L)
    def _(g):
        u_v[pl.ds(g * L, L)] = jnp.zeros((L,), jnp.float32)

    nrow = NPT // DH
    for tt in range(NT):
        pltpu.sync_copy(S_sh.at[pl.ds(tt * HR + wid * nrow, nrow)], tmp_v)

        @pl.loop(0, NPT // L)
        def _(g):
            s = pl.ds(g * L, L)
            u_v[s] = u_v[s] + tmp_v[g // (DH // L), pl.ds((g % (DH // L)) * L, L)]

    plsc.subcore_barrier()

    @pl.loop(0, NPT // L)
    def _(g):
        s = pl.ds(g * L, L)
        deg = u_v[s] + 1.0
        y = _rsqrt16(deg)
        y_v[s] = y
        rd_v[s] = y * deg
        u_v[s] = (1.0 - ALPHA) * y * y

    for ch in range(NUC):
        rb = nbase + ch * UCH
        pltpu.sync_copy(h0_hbm.at[pl.ds(coff + rb, UCH)], pbuf.at[0])

        @pl.loop(0, UCH)
        def _(r):
            yb = _bcast_row(y_v, ch * UCH + r)
            for i in range(DH // L):
                s = pl.ds(i * L, L)
                h0r = pbuf[0, r, s]
                cbuf[0, r, s] = (ALPHA * yb) * h0r
                pbuf[0, r, s] = yb * h0r

        pltpu.sync_copy(cbuf.at[0], c_hbm.at[pl.ds(coff + rb, UCH)])
        pltpu.sync_copy(pbuf.at[0], p_sh.at[pl.ds(rb, UCH)])
        pltpu.sync_copy(zbuf, S_sh.at[pl.ds(rb, UCH)])

    plsc.subcore_barrier()

    def i_start(j, b8):
        pltpu.async_copy(idxp_hbm.at[wid, j], pairbuf.at[b8], ip[b8])

    def i_wait(j, b8):
        pltpu.make_async_copy(idxp_hbm.at[wid, j], pairbuf.at[b8],
                              ip[b8]).wait()

    def g_start(b, b8):
        return pltpu.async_copy(p_sh.at[pairbuf.at[b8, 1]], rowbuf.at[b],
                                gs[b])

    def g_wait(b, b8):
        pltpu.make_async_copy(p_sh.at[pairbuf.at[b8, 1]], rowbuf.at[b],
                              gs[b]).wait()

    def s_start(b, b8):
        return pltpu.async_copy(rowbuf.at[b], S_sh.at[pairbuf.at[b8, 0]],
                                ss[b], add=True)

    def s_wait(b, b8):
        pltpu.make_async_copy(rowbuf.at[b], S_sh.at[pairbuf.at[b8, 0]],
                              ss[b]).wait()

    def upd_prefetch(ch, b):
        rb = nbase + ch * UCH
        pltpu.async_copy(S_sh.at[pl.ds(rb, UCH)], sbuf.at[b], us[b])
        pltpu.async_copy(p_sh.at[pl.ds(rb, UCH)], pbuf.at[b], up[b])
        pltpu.async_copy(c_hbm.at[pl.ds(coff + rb, UCH)], cbuf.at[b], uc[b])

    def upd_wait_in(ch, b):
        rb = nbase + ch * UCH
        pltpu.make_async_copy(S_sh.at[pl.ds(rb, UCH)], sbuf.at[b],
                              us[b]).wait()
        pltpu.make_async_copy(p_sh.at[pl.ds(rb, UCH)], pbuf.at[b],
                              up[b]).wait()
        pltpu.make_async_copy(c_hbm.at[pl.ds(coff + rb, UCH)], cbuf.at[b],
                              uc[b]).wait()

    def upd_wait_w(ch, b):
        rb = nbase + ch * UCH
        pltpu.make_async_copy(pbuf.at[b], p_sh.at[pl.ds(rb, UCH)],
                              uw[b]).wait()

    @pl.loop(0, K_PROP)
    def _(k):
        def slot(j, t, first, last):
            b = t % NBUF
            g_wait(b, t)
            s_start(b, t)
            if not (first and t < 2):
                s_wait((t - 2) % NBUF, (t - 2) % IBUF)
            if not (last and t >= IBUF - 2):
                i_wait(j + 2, (t + 2) % IBUF)
                g_start((t + 2) % NBUF, (t + 2) % IBUF)
            if not (last and t >= 2):
                i_start(j + 6, (t + 6) % IBUF)

        for jp in range(IBUF - 2):
            i_start(jp, jp)
        i_wait(0, 0)
        g_start(0, 0)
        i_wait(1, 1)
        g_start(1, 1)

        for t in range(IBUF):
            slot(t, t, True, False)

        @pl.loop(0, NCH // IBUF - 2)
        def _(jj):
            for t in range(IBUF):
                slot((jj + 1) * IBUF + t, t, False, False)

        for t in range(IBUF):
            slot(NCH - IBUF + t, t, False, True)

        s_wait((NCH - 2) % NBUF, (NCH - 2) % IBUF)
        s_wait((NCH - 1) % NBUF, (NCH - 1) % IBUF)

        plsc.subcore_barrier()

        upd_prefetch(0, 0)
        for ch in range(NUC):
            b = ch % 2
            upd_wait_in(ch, b)
            if ch + 1 < NUC:
                if ch >= 1:
                    upd_wait_w(ch - 1, 1 - b)
                upd_prefetch(ch + 1, 1 - b)

            @pl.loop(0, UCH)
            def _(r):
                ub = _bcast_row(u_v, ch * UCH + r)
                for i in range(DH // L):
                    s = pl.ds(i * L, L)
                    pbuf[b, r, s] = (ub * (sbuf[b, r, s] + pbuf[b, r, s])
                                     + cbuf[b, r, s])

            rb = nbase + ch * UCH
            pltpu.async_copy(pbuf.at[b], p_sh.at[pl.ds(rb, UCH)], uw[b])
            pltpu.async_copy(zbuf, S_sh.at[pl.ds(rb, UCH)], uz)

        upd_wait_w(NUC - 2, 1 - (NUC - 1) % 2)
        upd_wait_w(NUC - 1, (NUC - 1) % 2)
        for ch in range(NUC):
            rb = nbase + ch * UCH
            pltpu.make_async_copy(zbuf, S_sh.at[pl.ds(rb, UCH)], uz).wait()

        plsc.subcore_barrier()

    for ch in range(NUC):
        rb = nbase + ch * UCH
        pltpu.sync_copy(p_sh.at[pl.ds(rb, UCH)], pbuf.at[0])

        @pl.loop(0, UCH)
        def _(r):
            rdb = _bcast_row(rd_v, ch * UCH + r)
            for i in range(DH // L):
                s = pl.ds(i * L, L)
                pbuf[0, r, s] = rdb * pbuf[0, r, s]

        pltpu.sync_copy(pbuf.at[0], out_hbm.at[pl.ds(coff + rb, UCH)])


@functools.partial(
    pl.kernel,
    out_type=(
        jax.ShapeDtypeStruct((NC * NP, DH), jnp.float32),
        jax.ShapeDtypeStruct((NC * NP, DH), jnp.float32),
    ),
    mesh=plsc.VectorSubcoreMesh(
        core_axis_name="c", subcore_axis_name="s", num_cores=NC),
    compiler_params=pltpu.CompilerParams(
        needs_layout_passes=False, use_tc_tiling_on_sc=False),
    scratch_types=[
        pltpu.VMEM_SHARED((NP, DH), jnp.float32),
        pltpu.VMEM_SHARED((NP, DH), jnp.float32),
        pltpu.VMEM((IBUF, 2, CH), jnp.int32),
        pltpu.VMEM((NBUF, CH, DH), jnp.float32),
        pltpu.VMEM((HR, DH), jnp.float32),
        pltpu.VMEM((NPT // DH, DH), jnp.float32),
        pltpu.VMEM((NPT,), jnp.float32),
        pltpu.VMEM((NPT,), jnp.float32),
        pltpu.VMEM((NPT,), jnp.float32),
        pltpu.VMEM((2, UCH, DH), jnp.float32),
        pltpu.VMEM((2, UCH, DH), jnp.float32),
        pltpu.VMEM((2, UCH, DH), jnp.float32),
        pltpu.VMEM((UCH, DH), jnp.float32),
        [pltpu.SemaphoreType.DMA] * NBUF,
        [pltpu.SemaphoreType.DMA] * NBUF,
        [pltpu.SemaphoreType.DMA] * IBUF,
        [pltpu.SemaphoreType.DMA] * 2,
        [pltpu.SemaphoreType.DMA] * 2,
        [pltpu.SemaphoreType.DMA] * 2,
        [pltpu.SemaphoreType.DMA] * 2,
        pltpu.SemaphoreType.DMA,
    ],
)
def _sc_propagate(h0_hbm, idxp_hbm, out_hbm, c_hbm, *rest):
    _sc_body(h0_hbm, idxp_hbm, out_hbm, c_hbm, *rest)


def kernel(x, edge_index, W, b):
    src = edge_index[0].astype(jnp.int32)
    dst = edge_index[1].astype(jnp.int32)
    h0 = _h0_matmul(x, W.T, b.reshape(1, D))
    h0p = jnp.pad(h0, ((0, NP - N), (0, 0)))
    h0f = jnp.concatenate([h0p[:, :DH], h0p[:, DH:]], axis=0)
    srcp = jnp.pad(src.reshape(NT, EPT), ((0, 0), (0, EPT_P - EPT)),
                   constant_values=DUMMY_SRC).reshape(NT, NCH, 1, CH)
    dstp = jnp.pad(dst.reshape(NT, EPT), ((0, 0), (0, EPT_P - EPT)),
                   constant_values=DUMMY_DST).reshape(NT, NCH, 1, CH)
    idxp = jnp.concatenate([srcp, dstp], axis=2)
    out, _ = _sc_propagate(h0f, idxp)
    return jnp.concatenate([out[:N], out[NP:NP + N]], axis=1)


# speedup vs baseline: 1.6392x; 1.0140x over previous
import functools

import jax
import jax.numpy as jnp
from jax import lax
from jax.experimental import pallas as pl
from jax.experimental.pallas import tpu as pltpu
from jax.experimental.pallas import tpu_sc as plsc

N = 10000
D = 128
E = 320000
ALPHA = 0.1
K_PROP = 10

NC = 2
DH = D // NC
NT = 16
NP = 10240
NPT = NP // NT
EPT = E // NT
CH = 80
NBUF = 4
IBUF = 8
NCH = 256
EPT_P = NCH * CH
DUMMY_SRC = N
DUMMY_DST = N + 1
UCH = 32
NUC = NPT // UCH
L = 16
HR = NP // DH


def _mm_body(x_ref, w_ref, b_ref, o_ref):
    o_ref[...] = (
        jnp.dot(x_ref[...], w_ref[...], preferred_element_type=jnp.float32)
        + b_ref[...]
    )


def _h0_matmul(x, wt, b2):
    return pl.pallas_call(
        _mm_body,
        grid=(10,),
        in_specs=[
            pl.BlockSpec((N // 10, D), lambda i: (i, 0)),
            pl.BlockSpec((D, D), lambda i: (0, 0)),
            pl.BlockSpec((1, D), lambda i: (0, 0)),
        ],
        out_specs=pl.BlockSpec((N // 10, D), lambda i: (i, 0)),
        out_shape=jax.ShapeDtypeStruct((N, D), jnp.float32),
    )(x, wt, b2)


def _bcast_row(scal_ref, row):
    idx = jnp.full((L,), row, dtype=jnp.int32)
    return plsc.load_gather(scal_ref, [idx])


def _rsqrt16(d):
    i = plsc.bitcast(d, jnp.int32)
    y = plsc.bitcast(jnp.int32(0x5F3759DF) - (i >> 1), jnp.float32)
    for _ in range(3):
        y = y * (1.5 - 0.5 * d * y * y)
    return y


def _sc_body(h0_hbm, idxp_hbm, out_hbm, c_hbm,
             S_sh, p_sh,
             pairbuf, rowbuf, hist_v, tmp_v, u_v, y_v, rd_v,
             sbuf, pbuf, cbuf, zbuf,
             gs, ss, ip, us, up, uc, uw, uz):
    cid = lax.axis_index("c")
    wid = lax.axis_index("s")
    nbase = wid * NPT
    coff = cid * NP

    @pl.loop(0, HR)
    def _(g):
        for i in range(DH // L):
            hist_v[g, pl.ds(i * L, L)] = jnp.zeros((L,), jnp.float32)

    @pl.loop(0, UCH)
    def _(r):
        for i in range(DH // L):
            zbuf[r, pl.ds(i * L, L)] = jnp.zeros((L,), jnp.float32)

    ones = jnp.ones((L,), jnp.float32)

    @pl.loop(0, NCH // IBUF)
    def _(g):
        pltpu.sync_copy(idxp_hbm.at[wid, pl.ds(g * IBUF, IBUF)], pairbuf)
        for q in range(IBUF):
            for i in range(CH // L):
                sv = pairbuf[q, 0, pl.ds(i * L, L)]
                plsc.addupdate_scatter(hist_v, [sv >> 6, sv & 63], ones)

    pltpu.sync_copy(hist_v, S_sh.at[pl.ds(wid * HR, HR)])
    plsc.subcore_barrier()

    @pl.loop(0, NPT // L)
    def _(g):
        u_v[pl.ds(g * L, L)] = jnp.zeros((L,), jnp.float32)

    nrow = NPT // DH
    for tt in range(NT):
        pltpu.sync_copy(S_sh.at[pl.ds(tt * HR + wid * nrow, nrow)], tmp_v)

        @pl.loop(0, NPT // L)
        def _(g):
            s = pl.ds(g * L, L)
            u_v[s] = u_v[s] + tmp_v[g // (DH // L), pl.ds((g % (DH // L)) * L, L)]

    plsc.subcore_barrier()

    @pl.loop(0, NPT // L)
    def _(g):
        s = pl.ds(g * L, L)
        deg = u_v[s] + 1.0
        y = _rsqrt16(deg)
        y_v[s] = y
        rd_v[s] = y * deg
        u_v[s] = (1.0 - ALPHA) * y * y

    for ch in range(NUC):
        rb = nbase + ch * UCH
        pltpu.sync_copy(h0_hbm.at[pl.ds(coff + rb, UCH)], pbuf.at[0])

        @pl.loop(0, UCH)
        def _(r):
            yb = _bcast_row(y_v, ch * UCH + r)
            for i in range(DH // L):
                s = pl.ds(i * L, L)
                h0r = pbuf[0, r, s]
                cbuf[0, r, s] = (ALPHA * yb) * h0r
                pbuf[0, r, s] = yb * h0r

        pltpu.sync_copy(cbuf.at[0], c_hbm.at[pl.ds(coff + rb, UCH)])
        pltpu.sync_copy(pbuf.at[0], p_sh.at[pl.ds(rb, UCH)])
        pltpu.sync_copy(zbuf, S_sh.at[pl.ds(rb, UCH)])

    plsc.subcore_barrier()

    def i_start(j, b8):
        pltpu.async_copy(idxp_hbm.at[wid, j], pairbuf.at[b8], ip[b8])

    def i_wait(j, b8):
        pltpu.make_async_copy(idxp_hbm.at[wid, j], pairbuf.at[b8],
                              ip[b8]).wait()

    def g_start(b, b8):
        return pltpu.async_copy(p_sh.at[pairbuf.at[b8, 1]], rowbuf.at[b],
                                gs[b])

    def g_wait(b, b8):
        pltpu.make_async_copy(p_sh.at[pairbuf.at[b8, 1]], rowbuf.at[b],
                              gs[b]).wait()

    def s_start(b, b8):
        return pltpu.async_copy(rowbuf.at[b], S_sh.at[pairbuf.at[b8, 0]],
                                ss[b], add=True)

    def s_wait(b, b8):
        pltpu.make_async_copy(rowbuf.at[b], S_sh.at[pairbuf.at[b8, 0]],
                              ss[b]).wait()

    def upd_prefetch(ch, b):
        rb = nbase + ch * UCH
        pltpu.async_copy(S_sh.at[pl.ds(rb, UCH)], sbuf.at[b], us[b])
        pltpu.async_copy(p_sh.at[pl.ds(rb, UCH)], pbuf.at[b], up[b])
        pltpu.async_copy(c_hbm.at[pl.ds(coff + rb, UCH)], cbuf.at[b], uc[b])

    def upd_wait_in(ch, b):
        rb = nbase + ch * UCH
        pltpu.make_async_copy(S_sh.at[pl.ds(rb, UCH)], sbuf.at[b],
                              us[b]).wait()
        pltpu.make_async_copy(p_sh.at[pl.ds(rb, UCH)], pbuf.at[b],
                              up[b]).wait()
        pltpu.make_async_copy(c_hbm.at[pl.ds(coff + rb, UCH)], cbuf.at[b],
                              uc[b]).wait()

    def upd_wait_w(ch, b):
        rb = nbase + ch * UCH
        pltpu.make_async_copy(pbuf.at[b], p_sh.at[pl.ds(rb, UCH)],
                              uw[b]).wait()

    @pl.loop(0, K_PROP)
    def _(k):
        def slot(j, t, first, last):
            b = t % NBUF
            g_wait(b, t)
            s_start(b, t)
            if not (first and t < 2):
                s_wait((t - 2) % NBUF, (t - 2) % IBUF)
            if not (last and t >= IBUF - 2):
                i_wait(j + 2, (t + 2) % IBUF)
                g_start((t + 2) % NBUF, (t + 2) % IBUF)
            if not (last and t >= 2):
                i_start(j + 6, (t + 6) % IBUF)

        for jp in range(IBUF - 2):
            i_start(jp, jp)
        i_wait(0, 0)
        g_start(0, 0)
        i_wait(1, 1)
        g_start(1, 1)

        for t in range(IBUF):
            slot(t, t, True, False)

        @pl.loop(0, NCH // IBUF - 2)
        def _(jj):
            for t in range(IBUF):
                slot((jj + 1) * IBUF + t, t, False, False)

        for t in range(IBUF):
            slot(NCH - IBUF + t, t, False, True)

        s_wait((NCH - 2) % NBUF, (NCH - 2) % IBUF)
        s_wait((NCH - 1) % NBUF, (NCH - 1) % IBUF)

        plsc.subcore_barrier()

        upd_prefetch(0, 0)
        for ch in range(NUC):
            b = ch % 2
            upd_wait_in(ch, b)
            if ch + 1 < NUC:
                if ch >= 1:
                    upd_wait_w(ch - 1, 1 - b)
                upd_prefetch(ch + 1, 1 - b)

            @pl.loop(0, UCH)
            def _(r):
                ub = _bcast_row(u_v, ch * UCH + r)
                for i in range(DH // L):
                    s = pl.ds(i * L, L)
                    pbuf[b, r, s] = (ub * (sbuf[b, r, s] + pbuf[b, r, s])
                                     + cbuf[b, r, s])

            rb = nbase + ch * UCH
            pltpu.async_copy(pbuf.at[b], p_sh.at[pl.ds(rb, UCH)], uw[b])
            pltpu.async_copy(zbuf, S_sh.at[pl.ds(rb, UCH)], uz)

        upd_wait_w(NUC - 2, 1 - (NUC - 1) % 2)
        upd_wait_w(NUC - 1, (NUC - 1) % 2)
        for ch in range(NUC):
            rb = nbase + ch * UCH
            pltpu.make_async_copy(zbuf, S_sh.at[pl.ds(rb, UCH)], uz).wait()

        plsc.subcore_barrier()

    for ch in range(NUC):
        rb = nbase + ch * UCH
        pltpu.sync_copy(p_sh.at[pl.ds(rb, UCH)], pbuf.at[0])

        @pl.loop(0, UCH)
        def _(r):
            rdb = _bcast_row(rd_v, ch * UCH + r)
            for i in range(DH // L):
                s = pl.ds(i * L, L)
                pbuf[0, r, s] = rdb * pbuf[0, r, s]

        pltpu.sync_copy(pbuf.at[0], out_hbm.at[pl.ds(coff + rb, UCH)])


@functools.partial(
    pl.kernel,
    out_type=(
        jax.ShapeDtypeStruct((NC * NP, DH), jnp.float32),
        jax.ShapeDtypeStruct((NC * NP, DH), jnp.float32),
    ),
    mesh=plsc.VectorSubcoreMesh(
        core_axis_name="c", subcore_axis_name="s", num_cores=NC),
    compiler_params=pltpu.CompilerParams(
        needs_layout_passes=False, use_tc_tiling_on_sc=False),
    scratch_types=[
        pltpu.VMEM_SHARED((NP, DH), jnp.float32),
        pltpu.VMEM_SHARED((NP, DH), jnp.float32),
        pltpu.VMEM((IBUF, 2, CH), jnp.int32),
        pltpu.VMEM((NBUF, CH, DH), jnp.float32),
        pltpu.VMEM((HR, DH), jnp.float32),
        pltpu.VMEM((NPT // DH, DH), jnp.float32),
        pltpu.VMEM((NPT,), jnp.float32),
        pltpu.VMEM((NPT,), jnp.float32),
        pltpu.VMEM((NPT,), jnp.float32),
        pltpu.VMEM((2, UCH, DH), jnp.float32),
        pltpu.VMEM((2, UCH, DH), jnp.float32),
        pltpu.VMEM((2, UCH, DH), jnp.float32),
        pltpu.VMEM((UCH, DH), jnp.float32),
        [pltpu.SemaphoreType.DMA] * NBUF,
        [pltpu.SemaphoreType.DMA] * NBUF,
        [pltpu.SemaphoreType.DMA] * IBUF,
        [pltpu.SemaphoreType.DMA] * 2,
        [pltpu.SemaphoreType.DMA] * 2,
        [pltpu.SemaphoreType.DMA] * 2,
        [pltpu.SemaphoreType.DMA] * 2,
        pltpu.SemaphoreType.DMA,
    ],
)
def _sc_propagate(h0_hbm, idxp_hbm, out_hbm, c_hbm, *rest):
    _sc_body(h0_hbm, idxp_hbm, out_hbm, c_hbm, *rest)


def kernel(x, edge_index, W, b):
    src = edge_index[0].astype(jnp.int32)
    dst = edge_index[1].astype(jnp.int32)
    h0 = _h0_matmul(x, W.T, b.reshape(1, D))
    h0p = jnp.pad(h0, ((0, NP - N), (0, 0)))
    h0f = jnp.concatenate([h0p[:, :DH], h0p[:, DH:]], axis=0)
    srcp = jnp.pad(src.reshape(NT, EPT), ((0, 0), (0, EPT_P - EPT)),
                   constant_values=DUMMY_SRC).reshape(NT, NCH, 1, CH)
    dstp = jnp.pad(dst.reshape(NT, EPT), ((0, 0), (0, EPT_P - EPT)),
                   constant_values=DUMMY_DST).reshape(NT, NCH, 1, CH)
    idxp = jnp.concatenate([srcp, dstp], axis=2)
    out, _ = _sc_propagate(h0f, idxp)
    return jnp.concatenate([out[:N], out[NP:NP + N]], axis=1)
